# baseline (speedup 1.0000x reference)
import numpy as np

import jax
import jax.numpy as jnp
from jax import lax
from jax.experimental import pallas as pl
from jax.experimental.pallas import tpu as pltpu
from jax.experimental.pallas import tpu_sc as plsc

B = 4
NPOINT = 1024
N = 16384
K = 32
L = 16
NV = N // L
NSEG = 32
SEGV = NV // NSEG
NTEC = 32
CPT = (B * NPOINT) // NTEC
TPB = NTEC // B
CAP = N + 2 * L

F32_INF = np.float32(np.inf)
F32_NINF = np.float32(-np.inf)
I32_MAX = np.int32(2**31 - 1)


def _knn_body(xt, ct, out, xv, yv, zv, cxv, cyv, czv, distv, cand_d, cand_i,
              outbuf):
    wid = lax.axis_index("s") * 2 + lax.axis_index("c")
    b = wid // TPB
    c0 = (wid % TPB) * CPT

    pltpu.sync_copy(xt.at[pl.ds((b * 3 + 0) * N, N)], xv)
    pltpu.sync_copy(xt.at[pl.ds((b * 3 + 1) * N, N)], yv)
    pltpu.sync_copy(xt.at[pl.ds((b * 3 + 2) * N, N)], zv)
    pltpu.sync_copy(ct.at[pl.ds(((b * 3 + 0) * NPOINT + c0) * L, CPT * L)], cxv)
    pltpu.sync_copy(ct.at[pl.ds(((b * 3 + 1) * NPOINT + c0) * L, CPT * L)], cyv)
    pltpu.sync_copy(ct.at[pl.ds(((b * 3 + 2) * NPOINT + c0) * L, CPT * L)], czv)

    iota = lax.iota(jnp.int32, L)

    def center_body(ci, _):
        cx = cxv[pl.ds(ci * L, L)]
        cy = cyv[pl.ds(ci * L, L)]
        cz = czv[pl.ds(ci * L, L)]

        def seg_body(s, t):
            def vreg_body(j, m):
                off = (s * SEGV + j) * L
                x = xv[pl.ds(off, L)]
                y = yv[pl.ds(off, L)]
                z = zv[pl.ds(off, L)]
                dx = cx - x
                dy = cy - y
                dz = cz - z
                d = (dx * dx + dy * dy) + dz * dz
                distv[pl.ds(off, L)] = d
                return jnp.minimum(m, d)

            m = lax.fori_loop(0, SEGV, vreg_body,
                              jnp.full((L,), F32_INF, jnp.float32), unroll=4)
            return jnp.maximum(t, jnp.min(m))

        t = lax.fori_loop(0, NSEG, seg_body, jnp.float32(F32_NINF))

        def p2_body(i, off):
            o16 = i * L
            d = distv[pl.ds(o16, L)]
            sel = d <= t
            cnt = plsc.all_reduce_population_count(sel)[0]

            def do_store(off):
                pc = plsc.cumsum(sel.astype(jnp.int32))
                dest = off + pc - 1
                plsc.store_scatter(cand_d, [dest], d, mask=sel)
                plsc.store_scatter(cand_i, [dest], iota + o16, mask=sel)
                return off + cnt

            return lax.cond(cnt > 0, do_store, lambda o: o, off)

        c = lax.fori_loop(0, NV, p2_body, jnp.int32(0), unroll=2)
        nv = (c + L - 1) // L

        def ext_body(j, state):
            pd, pi, ov0, ov1 = state

            def scan_body(v, bs):
                bd, bi = bs
                o16 = v * L
                d = cand_d[pl.ds(o16, L)]
                ii = cand_i[pl.ds(o16, L)]
                ok = ((iota + o16) < c) & ((d > pd) | ((d == pd) & (ii > pi)))
                d = jnp.where(ok, d, F32_INF)
                ii = jnp.where(ok, ii, I32_MAX)
                better = (d < bd) | ((d == bd) & (ii < bi))
                return (jnp.where(better, d, bd), jnp.where(better, ii, bi))

            bd, bi = lax.fori_loop(0, nv, scan_body,
                                   (jnp.full((L,), F32_INF, jnp.float32),
                                    jnp.full((L,), I32_MAX, jnp.int32)))
            dmin = jnp.min(bd)
            imin = jnp.min(jnp.where(bd == dmin, bi, I32_MAX))
            ov0 = jnp.where((j < L) & (iota == j), imin, ov0)
            ov1 = jnp.where((j >= L) & (iota == j - L), imin, ov1)
            return (dmin, imin, ov0, ov1)

        zero16 = jnp.zeros((L,), jnp.int32)
        _, _, ov0, ov1 = lax.fori_loop(
            0, K, ext_body,
            (jnp.float32(F32_NINF), jnp.int32(-1), zero16, zero16))
        outbuf[pl.ds(ci * K, L)] = ov0
        outbuf[pl.ds(ci * K + L, L)] = ov1
        return 0

    lax.fori_loop(0, CPT, center_body, 0)
    pltpu.sync_copy(outbuf, out.at[pl.ds(wid * (CPT * K), CPT * K)])


@jax.jit
def _knn(xt, ct):
    f = pl.kernel(
        _knn_body,
        out_type=jax.ShapeDtypeStruct((B * NPOINT * K,), jnp.int32),
        mesh=plsc.VectorSubcoreMesh(core_axis_name="c", subcore_axis_name="s"),
        compiler_params=pltpu.CompilerParams(needs_layout_passes=False),
        scratch_types=[
            pltpu.VMEM((N,), jnp.float32),
            pltpu.VMEM((N,), jnp.float32),
            pltpu.VMEM((N,), jnp.float32),
            pltpu.VMEM((CPT * L,), jnp.float32),
            pltpu.VMEM((CPT * L,), jnp.float32),
            pltpu.VMEM((CPT * L,), jnp.float32),
            pltpu.VMEM((N,), jnp.float32),
            pltpu.VMEM((CAP,), jnp.float32),
            pltpu.VMEM((CAP,), jnp.int32),
            pltpu.VMEM((CPT * K,), jnp.int32),
        ],
    )
    return f(xt, ct)


def kernel(xyz, center):
    xt = jnp.transpose(xyz, (0, 2, 1)).reshape(B * 3 * N)
    ct = jnp.repeat(jnp.transpose(center, (0, 2, 1)).reshape(B * 3 * NPOINT), L)
    return _knn(xt, ct).reshape(B, NPOINT, K)


# speedup vs baseline: 4.4368x; 4.4368x over previous
import numpy as np

import jax
import jax.numpy as jnp
from jax import lax
from jax.experimental import pallas as pl
from jax.experimental.pallas import tpu as pltpu
from jax.experimental.pallas import tpu_sc as plsc

B = 4
NPOINT = 1024
N = 16384
K = 32
L = 16
NV = N // L
NSEG = 32
SEGV = NV // NSEG
NTEC = 32
CPT = (B * NPOINT) // NTEC
TPB = NTEC // B
CAP = N + 2 * L

F32_INF = np.float32(np.inf)
F32_NINF = np.float32(-np.inf)
I32_MAX = np.int32(2**31 - 1)


def _knn_body(xt, ct, out, xv, yv, zv, cxv, cyv, czv, distv, cand_d, cand_i,
              outbuf):
    wid = lax.axis_index("s") * 2 + lax.axis_index("c")
    b = wid // TPB
    c0 = (wid % TPB) * CPT

    pltpu.sync_copy(xt.at[pl.ds((b * 3 + 0) * N, N)], xv)
    pltpu.sync_copy(xt.at[pl.ds((b * 3 + 1) * N, N)], yv)
    pltpu.sync_copy(xt.at[pl.ds((b * 3 + 2) * N, N)], zv)
    pltpu.sync_copy(ct.at[pl.ds(((b * 3 + 0) * NPOINT + c0) * L, CPT * L)], cxv)
    pltpu.sync_copy(ct.at[pl.ds(((b * 3 + 1) * NPOINT + c0) * L, CPT * L)], cyv)
    pltpu.sync_copy(ct.at[pl.ds(((b * 3 + 2) * NPOINT + c0) * L, CPT * L)], czv)

    iota = lax.iota(jnp.int32, L)

    def center_body(ci, _):
        cx = cxv[pl.ds(ci * L, L)]
        cy = cyv[pl.ds(ci * L, L)]
        cz = czv[pl.ds(ci * L, L)]

        def seg_body(s, t):
            @plsc.parallel_loop(s * SEGV, (s + 1) * SEGV, unroll=4,
                                carry=jnp.full((L,), F32_INF, jnp.float32))
            def p1_loop(j, m):
                off = j * L
                x = xv[pl.ds(off, L)]
                y = yv[pl.ds(off, L)]
                z = zv[pl.ds(off, L)]
                dx = cx - x
                dy = cy - y
                dz = cz - z
                d = (dx * dx + dy * dy) + dz * dz
                distv[pl.ds(off, L)] = d
                return jnp.minimum(m, d)

            return jnp.maximum(t, jnp.min(p1_loop))

        t = lax.fori_loop(0, NSEG, seg_body, jnp.float32(F32_NINF))

        @plsc.parallel_loop(0, NV, unroll=2, carry=jnp.int32(0))
        def p2_loop(i, off):
            o16 = i * L
            d = distv[pl.ds(o16, L)]
            sel = d <= t
            cnt = plsc.all_reduce_population_count(sel)[0]
            pc = plsc.cumsum(sel.astype(jnp.int32))
            dest = off + pc - 1
            plsc.store_scatter(cand_d, [dest], d, mask=sel)
            plsc.store_scatter(cand_i, [dest], iota + o16, mask=sel)
            return off + cnt

        c = p2_loop
        nv = (c + L - 1) // L

        def ext_body(j, state):
            pd, pi, ov0, ov1 = state

            def scan_body(v, bs):
                bd, bi = bs
                o16 = v * L
                d = cand_d[pl.ds(o16, L)]
                ii = cand_i[pl.ds(o16, L)]
                ok = ((iota + o16) < c) & ((d > pd) | ((d == pd) & (ii > pi)))
                d = jnp.where(ok, d, F32_INF)
                ii = jnp.where(ok, ii, I32_MAX)
                better = (d < bd) | ((d == bd) & (ii < bi))
                return (jnp.where(better, d, bd), jnp.where(better, ii, bi))

            bd, bi = lax.fori_loop(0, nv, scan_body,
                                   (jnp.full((L,), F32_INF, jnp.float32),
                                    jnp.full((L,), I32_MAX, jnp.int32)))
            dmin = jnp.min(bd)
            imin = jnp.min(jnp.where(bd == dmin, bi, I32_MAX))
            ov0 = jnp.where((j < L) & (iota == j), imin, ov0)
            ov1 = jnp.where((j >= L) & (iota == j - L), imin, ov1)
            return (dmin, imin, ov0, ov1)

        zero16 = jnp.zeros((L,), jnp.int32)
        _, _, ov0, ov1 = lax.fori_loop(
            0, K, ext_body,
            (jnp.float32(F32_NINF), jnp.int32(-1), zero16, zero16))
        outbuf[pl.ds(ci * K, L)] = ov0
        outbuf[pl.ds(ci * K + L, L)] = ov1
        return 0

    lax.fori_loop(0, CPT, center_body, 0)
    pltpu.sync_copy(outbuf, out.at[pl.ds(wid * (CPT * K), CPT * K)])


@jax.jit
def _knn(xt, ct):
    f = pl.kernel(
        _knn_body,
        out_type=jax.ShapeDtypeStruct((B * NPOINT * K,), jnp.int32),
        mesh=plsc.VectorSubcoreMesh(core_axis_name="c", subcore_axis_name="s"),
        compiler_params=pltpu.CompilerParams(needs_layout_passes=False),
        scratch_types=[
            pltpu.VMEM((N,), jnp.float32),
            pltpu.VMEM((N,), jnp.float32),
            pltpu.VMEM((N,), jnp.float32),
            pltpu.VMEM((CPT * L,), jnp.float32),
            pltpu.VMEM((CPT * L,), jnp.float32),
            pltpu.VMEM((CPT * L,), jnp.float32),
            pltpu.VMEM((N,), jnp.float32),
            pltpu.VMEM((CAP,), jnp.float32),
            pltpu.VMEM((CAP,), jnp.int32),
            pltpu.VMEM((CPT * K,), jnp.int32),
        ],
    )
    return f(xt, ct)


def kernel(xyz, center):
    xt = jnp.transpose(xyz, (0, 2, 1)).reshape(B * 3 * N)
    ct = jnp.repeat(jnp.transpose(center, (0, 2, 1)).reshape(B * 3 * NPOINT), L)
    return _knn(xt, ct).reshape(B, NPOINT, K)


# speedup vs baseline: 6.8864x; 1.5521x over previous
import numpy as np

import jax
import jax.numpy as jnp
from jax import lax
from jax.experimental import pallas as pl
from jax.experimental.pallas import tpu as pltpu
from jax.experimental.pallas import tpu_sc as plsc

B = 4
NPOINT = 1024
N = 16384
K = 32
L = 16
NV = N // L
NTEC = 32
CPT = (B * NPOINT) // NTEC
TPB = NTEC // B
CAP = N + 2 * L

F32_INF = np.float32(np.inf)
F32_NINF = np.float32(-np.inf)
I32_MAX = np.int32(2**31 - 1)


def _rev(x):
    return lax.rev(x, (0,))


def _lexminmax(ak, av, bk, bv):
    m = (ak < bk) | ((ak == bk) & (av < bv))
    return (jnp.where(m, ak, bk), jnp.where(m, av, bv),
            jnp.where(m, bk, ak), jnp.where(m, bv, av))


def _knn_body(xt, ct, out, xv, yv, zv, cxv, cyv, czv, distv, cand_d, cand_i,
              outbuf):
    wid = lax.axis_index("s") * 2 + lax.axis_index("c")
    b = wid // TPB
    c0 = (wid % TPB) * CPT

    pltpu.sync_copy(xt.at[pl.ds((b * 3 + 0) * N, N)], xv)
    pltpu.sync_copy(xt.at[pl.ds((b * 3 + 1) * N, N)], yv)
    pltpu.sync_copy(xt.at[pl.ds((b * 3 + 2) * N, N)], zv)
    pltpu.sync_copy(ct.at[pl.ds(((b * 3 + 0) * NPOINT + c0) * L, CPT * L)], cxv)
    pltpu.sync_copy(ct.at[pl.ds(((b * 3 + 1) * NPOINT + c0) * L, CPT * L)], cyv)
    pltpu.sync_copy(ct.at[pl.ds(((b * 3 + 2) * NPOINT + c0) * L, CPT * L)], czv)

    iota = lax.iota(jnp.int32, L)
    inf16 = jnp.full((L,), F32_INF, jnp.float32)
    imax16 = jnp.full((L,), I32_MAX, jnp.int32)

    def center_body(ci, _):
        cx = cxv[pl.ds(ci * L, L)]
        cy = cyv[pl.ds(ci * L, L)]
        cz = czv[pl.ds(ci * L, L)]

        @plsc.parallel_loop(0, NV, step=4, unroll=2,
                            carry=(inf16, inf16, inf16, inf16))
        def p1_loop(j, accs):
            new = []
            for r, a in enumerate(accs):
                off = (j + r) * L
                x = xv[pl.ds(off, L)]
                y = yv[pl.ds(off, L)]
                z = zv[pl.ds(off, L)]
                dx = cx - x
                dy = cy - y
                dz = cz - z
                d = (dx * dx + dy * dy) + dz * dz
                distv[pl.ds(off, L)] = d
                new.append(jnp.minimum(a, d))
            return tuple(new)

        a0, a1, a2, a3 = p1_loop

        s0, s1 = jnp.sort(a0), jnp.sort(a1)
        s2, s3 = jnp.sort(a2), jnp.sort(a3)
        r1, r3 = _rev(s1), _rev(s3)
        p0 = jnp.sort(jnp.minimum(s0, r1))
        p1 = jnp.sort(jnp.maximum(s0, r1))
        q0 = jnp.sort(jnp.minimum(s2, r3))
        q1 = jnp.sort(jnp.maximum(s2, r3))
        lo0 = jnp.minimum(p0, _rev(q1))
        lo1 = jnp.minimum(p1, _rev(q0))
        t = jnp.max(jnp.maximum(lo0, lo1))

        @plsc.parallel_loop(0, NV, unroll=2, carry=jnp.int32(0))
        def p2_loop(i, off):
            o16 = i * L
            d = distv[pl.ds(o16, L)]
            sel = d <= t
            cnt = plsc.all_reduce_population_count(sel)[0]
            pc = plsc.cumsum(sel.astype(jnp.int32))
            dest = off + pc - 1
            plsc.store_scatter(cand_d, [dest], d, mask=sel)
            plsc.store_scatter(cand_i, [dest], iota + o16, mask=sel)
            return off + cnt

        c = p2_loop
        nv = (c + L - 1) // L

        def scan_body(v, st):
            a0k, a0v, a1k, a1v = st
            o16 = v * L
            d = cand_d[pl.ds(o16, L)]
            ii = cand_i[pl.ds(o16, L)]
            valid = (iota + o16) < c
            d = jnp.where(valid, d, F32_INF)
            ii = jnp.where(valid, ii, I32_MAX)
            sk, sv = plsc.sort_key_val(d, ii)
            mk, mv, _, _ = _lexminmax(a1k, a1v, _rev(sk), _rev(sv))
            mk, mv = plsc.sort_key_val(mk, mv)
            lok, lov, hik, hiv = _lexminmax(a0k, a0v, _rev(mk), _rev(mv))
            a0k, a0v = plsc.sort_key_val(lok, lov)
            a1k, a1v = plsc.sort_key_val(hik, hiv)
            return (a0k, a0v, a1k, a1v)

        _, ov0, _, ov1 = lax.fori_loop(0, nv, scan_body,
                                       (inf16, imax16, inf16, imax16))
        outbuf[pl.ds(ci * K, L)] = ov0
        outbuf[pl.ds(ci * K + L, L)] = ov1
        return 0

    lax.fori_loop(0, CPT, center_body, 0)
    pltpu.sync_copy(outbuf, out.at[pl.ds(wid * (CPT * K), CPT * K)])


@jax.jit
def _knn(xt, ct):
    f = pl.kernel(
        _knn_body,
        out_type=jax.ShapeDtypeStruct((B * NPOINT * K,), jnp.int32),
        mesh=plsc.VectorSubcoreMesh(core_axis_name="c", subcore_axis_name="s"),
        compiler_params=pltpu.CompilerParams(needs_layout_passes=False),
        scratch_types=[
            pltpu.VMEM((N,), jnp.float32),
            pltpu.VMEM((N,), jnp.float32),
            pltpu.VMEM((N,), jnp.float32),
            pltpu.VMEM((CPT * L,), jnp.float32),
            pltpu.VMEM((CPT * L,), jnp.float32),
            pltpu.VMEM((CPT * L,), jnp.float32),
            pltpu.VMEM((N,), jnp.float32),
            pltpu.VMEM((CAP,), jnp.float32),
            pltpu.VMEM((CAP,), jnp.int32),
            pltpu.VMEM((CPT * K,), jnp.int32),
        ],
    )
    return f(xt, ct)


def kernel(xyz, center):
    xt = jnp.transpose(xyz, (0, 2, 1)).reshape(B * 3 * N)
    ct = jnp.repeat(jnp.transpose(center, (0, 2, 1)).reshape(B * 3 * NPOINT), L)
    return _knn(xt, ct).reshape(B, NPOINT, K)


# speedup vs baseline: 8.6810x; 1.2606x over previous
import numpy as np

import jax
import jax.numpy as jnp
from jax import lax
from jax.experimental import pallas as pl
from jax.experimental.pallas import tpu as pltpu
from jax.experimental.pallas import tpu_sc as plsc

B = 4
NPOINT = 1024
N = 16384
K = 32
L = 16
NV = N // L
NTEC = 32
CPT = (B * NPOINT) // NTEC
TPB = NTEC // B
CAP = N + 2 * L

F32_INF = np.float32(np.inf)
F32_NINF = np.float32(-np.inf)
I32_MAX = np.int32(2**31 - 1)


def _rev(x):
    return lax.rev(x, (0,))


def _lexminmax(ak, av, bk, bv):
    m = (ak < bk) | ((ak == bk) & (av < bv))
    return (jnp.where(m, ak, bk), jnp.where(m, av, bv),
            jnp.where(m, bk, ak), jnp.where(m, bv, av))


def _knn_body(xt, ct, out, xv, yv, zv, cxv, cyv, czv, distv, cand_d, cand_i,
              outbuf):
    wid = lax.axis_index("s") * 2 + lax.axis_index("c")
    b = wid // TPB
    c0 = (wid % TPB) * CPT

    pltpu.sync_copy(xt.at[pl.ds((b * 3 + 0) * N, N)], xv)
    pltpu.sync_copy(xt.at[pl.ds((b * 3 + 1) * N, N)], yv)
    pltpu.sync_copy(xt.at[pl.ds((b * 3 + 2) * N, N)], zv)
    pltpu.sync_copy(ct.at[pl.ds(((b * 3 + 0) * NPOINT + c0) * L, CPT * L)], cxv)
    pltpu.sync_copy(ct.at[pl.ds(((b * 3 + 1) * NPOINT + c0) * L, CPT * L)], cyv)
    pltpu.sync_copy(ct.at[pl.ds(((b * 3 + 2) * NPOINT + c0) * L, CPT * L)], czv)

    iota = lax.iota(jnp.int32, L)
    inf16 = jnp.full((L,), F32_INF, jnp.float32)
    imax16 = jnp.full((L,), I32_MAX, jnp.int32)

    def center_body(ci, _):
        cx = cxv[pl.ds(ci * L, L)]
        cy = cyv[pl.ds(ci * L, L)]
        cz = czv[pl.ds(ci * L, L)]

        @plsc.parallel_loop(0, NV, step=4, unroll=2,
                            carry=(inf16, inf16, inf16, inf16))
        def p1_loop(j, accs):
            new = []
            for r, a in enumerate(accs):
                off = (j + r) * L
                x = xv[pl.ds(off, L)]
                y = yv[pl.ds(off, L)]
                z = zv[pl.ds(off, L)]
                dx = cx - x
                dy = cy - y
                dz = cz - z
                d = (dx * dx + dy * dy) + dz * dz
                distv[pl.ds(off, L)] = d
                new.append(jnp.minimum(a, d))
            return tuple(new)

        a0, a1, a2, a3 = p1_loop

        s0, s1 = jnp.sort(a0), jnp.sort(a1)
        s2, s3 = jnp.sort(a2), jnp.sort(a3)
        r1, r3 = _rev(s1), _rev(s3)
        p0 = jnp.sort(jnp.minimum(s0, r1))
        p1 = jnp.sort(jnp.maximum(s0, r1))
        q0 = jnp.sort(jnp.minimum(s2, r3))
        q1 = jnp.sort(jnp.maximum(s2, r3))
        lo0 = jnp.minimum(p0, _rev(q1))
        lo1 = jnp.minimum(p1, _rev(q0))
        t = jnp.max(jnp.maximum(lo0, lo1))

        @plsc.parallel_loop(0, NV, unroll=4, carry=jnp.int32(0))
        def p2_loop(i, off):
            o16 = i * L
            d = distv[pl.ds(o16, L)]
            sel = d <= t
            cnt = plsc.all_reduce_population_count(sel)[0]
            plsc.store_compressed(cand_d.at[pl.ds(off, L)], d, mask=sel)
            plsc.store_compressed(cand_i.at[pl.ds(off, L)], iota + o16,
                                  mask=sel)
            return off + cnt

        c = p2_loop
        nv = (c + L - 1) // L

        def scan_body(v, st):
            a0k, a0v, a1k, a1v = st
            o16 = v * L
            d = cand_d[pl.ds(o16, L)]
            ii = cand_i[pl.ds(o16, L)]
            valid = (iota + o16) < c
            d = jnp.where(valid, d, F32_INF)
            ii = jnp.where(valid, ii, I32_MAX)
            sk, sv = plsc.sort_key_val(d, ii)
            mk, mv, _, _ = _lexminmax(a1k, a1v, _rev(sk), _rev(sv))
            mk, mv = plsc.sort_key_val(mk, mv)
            lok, lov, hik, hiv = _lexminmax(a0k, a0v, _rev(mk), _rev(mv))
            a0k, a0v = plsc.sort_key_val(lok, lov)
            a1k, a1v = plsc.sort_key_val(hik, hiv)
            return (a0k, a0v, a1k, a1v)

        _, ov0, _, ov1 = lax.fori_loop(0, nv, scan_body,
                                       (inf16, imax16, inf16, imax16))
        outbuf[pl.ds(ci * K, L)] = ov0
        outbuf[pl.ds(ci * K + L, L)] = ov1
        return 0

    lax.fori_loop(0, CPT, center_body, 0)
    pltpu.sync_copy(outbuf, out.at[pl.ds(wid * (CPT * K), CPT * K)])


@jax.jit
def _knn(xt, ct):
    f = pl.kernel(
        _knn_body,
        out_type=jax.ShapeDtypeStruct((B * NPOINT * K,), jnp.int32),
        mesh=plsc.VectorSubcoreMesh(core_axis_name="c", subcore_axis_name="s"),
        compiler_params=pltpu.CompilerParams(needs_layout_passes=False),
        scratch_types=[
            pltpu.VMEM((N,), jnp.float32),
            pltpu.VMEM((N,), jnp.float32),
            pltpu.VMEM((N,), jnp.float32),
            pltpu.VMEM((CPT * L,), jnp.float32),
            pltpu.VMEM((CPT * L,), jnp.float32),
            pltpu.VMEM((CPT * L,), jnp.float32),
            pltpu.VMEM((N,), jnp.float32),
            pltpu.VMEM((CAP,), jnp.float32),
            pltpu.VMEM((CAP,), jnp.int32),
            pltpu.VMEM((CPT * K,), jnp.int32),
        ],
    )
    return f(xt, ct)


def kernel(xyz, center):
    xt = jnp.transpose(xyz, (0, 2, 1)).reshape(B * 3 * N)
    ct = jnp.repeat(jnp.transpose(center, (0, 2, 1)).reshape(B * 3 * NPOINT), L)
    return _knn(xt, ct).reshape(B, NPOINT, K)


# speedup vs baseline: 9.3840x; 1.0810x over previous
import numpy as np

import jax
import jax.numpy as jnp
from jax import lax
from jax.experimental import pallas as pl
from jax.experimental.pallas import tpu as pltpu
from jax.experimental.pallas import tpu_sc as plsc

B = 4
NPOINT = 1024
N = 16384
K = 32
L = 16
NV = N // L
NTEC = 32
CPT = (B * NPOINT) // NTEC
TPB = NTEC // B
CAP = N + 2 * L

F32_INF = np.float32(np.inf)
F32_NINF = np.float32(-np.inf)
I32_MAX = np.int32(2**31 - 1)


def _rev(x):
    return lax.rev(x, (0,))


def _lexminmax(ak, av, bk, bv):
    m = (ak < bk) | ((ak == bk) & (av < bv))
    return (jnp.where(m, ak, bk), jnp.where(m, av, bv),
            jnp.where(m, bk, ak), jnp.where(m, bv, av))


def _knn_body(xt, ct, out, xv, yv, zv, cxv, cyv, czv, distv, cand_i,
              outbuf):
    wid = lax.axis_index("s") * 2 + lax.axis_index("c")
    b = wid // TPB
    c0 = (wid % TPB) * CPT

    pltpu.sync_copy(xt.at[pl.ds((b * 3 + 0) * N, N)], xv)
    pltpu.sync_copy(xt.at[pl.ds((b * 3 + 1) * N, N)], yv)
    pltpu.sync_copy(xt.at[pl.ds((b * 3 + 2) * N, N)], zv)
    pltpu.sync_copy(ct.at[pl.ds(((b * 3 + 0) * NPOINT + c0) * L, CPT * L)], cxv)
    pltpu.sync_copy(ct.at[pl.ds(((b * 3 + 1) * NPOINT + c0) * L, CPT * L)], cyv)
    pltpu.sync_copy(ct.at[pl.ds(((b * 3 + 2) * NPOINT + c0) * L, CPT * L)], czv)

    iota = lax.iota(jnp.int32, L)
    inf16 = jnp.full((L,), F32_INF, jnp.float32)
    imax16 = jnp.full((L,), I32_MAX, jnp.int32)

    def center_body(ci, _):
        cx = cxv[pl.ds(ci * L, L)]
        cy = cyv[pl.ds(ci * L, L)]
        cz = czv[pl.ds(ci * L, L)]

        @plsc.parallel_loop(0, NV, step=4, unroll=2,
                            carry=(inf16, inf16, inf16, inf16))
        def p1_loop(j, accs):
            new = []
            for r, a in enumerate(accs):
                off = (j + r) * L
                x = xv[pl.ds(off, L)]
                y = yv[pl.ds(off, L)]
                z = zv[pl.ds(off, L)]
                dx = cx - x
                dy = cy - y
                dz = cz - z
                d = (dx * dx + dy * dy) + dz * dz
                distv[pl.ds(off, L)] = d
                new.append(jnp.minimum(a, d))
            return tuple(new)

        a0, a1, a2, a3 = p1_loop

        s0, s1 = jnp.sort(a0), jnp.sort(a1)
        s2, s3 = jnp.sort(a2), jnp.sort(a3)
        r1, r3 = _rev(s1), _rev(s3)
        p0 = jnp.sort(jnp.minimum(s0, r1))
        p1 = jnp.sort(jnp.maximum(s0, r1))
        q0 = jnp.sort(jnp.minimum(s2, r3))
        q1 = jnp.sort(jnp.maximum(s2, r3))
        lo0 = jnp.minimum(p0, _rev(q1))
        lo1 = jnp.minimum(p1, _rev(q0))
        t = jnp.max(jnp.maximum(lo0, lo1))

        @plsc.parallel_loop(0, NV, unroll=4, carry=jnp.int32(0))
        def p2_loop(i, off):
            o16 = i * L
            d = distv[pl.ds(o16, L)]
            sel = d <= t
            cnt = plsc.all_reduce_population_count(sel)[0]
            plsc.store_compressed(cand_i.at[pl.ds(off, L)], iota + o16,
                                  mask=sel)
            return off + cnt

        c = p2_loop
        nv = (c + L - 1) // L

        def scan_body(v, st):
            a0k, a0v, a1k, a1v = st
            o16 = v * L
            ii = cand_i[pl.ds(o16, L)]
            valid = (iota + o16) < c
            d = plsc.load_gather(distv, [ii], mask=valid)
            d = jnp.where(valid, d, F32_INF)
            ii = jnp.where(valid, ii, I32_MAX)
            sk, sv = plsc.sort_key_val(d, ii)
            mk, mv, _, _ = _lexminmax(a1k, a1v, _rev(sk), _rev(sv))
            mk, mv = plsc.sort_key_val(mk, mv)
            lok, lov, hik, hiv = _lexminmax(a0k, a0v, _rev(mk), _rev(mv))
            a0k, a0v = plsc.sort_key_val(lok, lov)
            a1k, a1v = plsc.sort_key_val(hik, hiv)
            return (a0k, a0v, a1k, a1v)

        _, ov0, _, ov1 = lax.fori_loop(0, nv, scan_body,
                                       (inf16, imax16, inf16, imax16))
        outbuf[pl.ds(ci * K, L)] = ov0
        outbuf[pl.ds(ci * K + L, L)] = ov1
        return 0

    lax.fori_loop(0, CPT, center_body, 0)
    pltpu.sync_copy(outbuf, out.at[pl.ds(wid * (CPT * K), CPT * K)])


@jax.jit
def _knn(xt, ct):
    f = pl.kernel(
        _knn_body,
        out_type=jax.ShapeDtypeStruct((B * NPOINT * K,), jnp.int32),
        mesh=plsc.VectorSubcoreMesh(core_axis_name="c", subcore_axis_name="s"),
        compiler_params=pltpu.CompilerParams(needs_layout_passes=False),
        scratch_types=[
            pltpu.VMEM((N,), jnp.float32),
            pltpu.VMEM((N,), jnp.float32),
            pltpu.VMEM((N,), jnp.float32),
            pltpu.VMEM((CPT * L,), jnp.float32),
            pltpu.VMEM((CPT * L,), jnp.float32),
            pltpu.VMEM((CPT * L,), jnp.float32),
            pltpu.VMEM((N,), jnp.float32),
            pltpu.VMEM((CAP,), jnp.int32),
            pltpu.VMEM((CPT * K,), jnp.int32),
        ],
    )
    return f(xt, ct)


def kernel(xyz, center):
    xt = jnp.transpose(xyz, (0, 2, 1)).reshape(B * 3 * N)
    ct = jnp.repeat(jnp.transpose(center, (0, 2, 1)).reshape(B * 3 * NPOINT), L)
    return _knn(xt, ct).reshape(B, NPOINT, K)
